# baseline (speedup 1.0000x reference)
import functools

import jax
import jax.numpy as jnp
from jax import lax
from jax.experimental import pallas as pl
from jax.experimental.pallas import tpu as pltpu
from jax.experimental.pallas import tpu_sc as plsc

NC, NS, L = 2, 16, 16
NW = NC * NS
B, D, V = 16384, 32, 1000001
BPW = B // NW
CH = 128
NCH = BPW // CH

VR = 1000000
NT = VR // CH
TAIL = VR - NT * CH
TPS = -(-NT // 16)
FLAT = V * D

_mesh = plsc.VectorSubcoreMesh(
    core_axis_name="c", subcore_axis_name="s", num_cores=NC, num_subcores=NS
)


@functools.partial(
    pl.kernel,
    out_type=(jax.ShapeDtypeStruct((FLAT,), jnp.float32),
              jax.ShapeDtypeStruct((FLAT,), jnp.float32)),
    mesh=_mesh,
    scratch_types=[
        pltpu.VMEM((D, CH), jnp.float32),
        pltpu.VMEM((D, CH), jnp.float32),
        pltpu.VMEM((CH * D,), jnp.float32),
        pltpu.VMEM((CH * D,), jnp.float32),
        pltpu.VMEM((TAIL * D,), jnp.float32),
        pltpu.SemaphoreType.DMA,
        pltpu.SemaphoreType.DMA,
    ],
    compiler_params=pltpu.CompilerParams(needs_layout_passes=False),
)
def _relayout(utT_hbm, itT_hbm, utail_hbm, itail_hbm, uflat_hbm, iflat_hbm,
              wbufA, wbufB, rbuf0, rbuf1, tbuf, rsem, wsem):
    wid = lax.axis_index("s") * NC + lax.axis_index("c")
    tbl = wid % 2
    sub = wid // 2
    t0 = sub * TPS

    lanes = lax.iota(jnp.int32, L)

    def read(t, wbuf, width):
        @pl.when(tbl == 0)
        def _():
            pltpu.async_copy(utT_hbm.at[:, pl.ds(t * CH, width)], wbuf, rsem)
        @pl.when(tbl == 1)
        def _():
            pltpu.async_copy(itT_hbm.at[:, pl.ds(t * CH, width)], wbuf, rsem)

    def rwait(wbuf):
        pltpu.make_async_copy(utT_hbm.at[:, pl.ds(0, CH)], wbuf, rsem).wait()

    def shuffle(wbuf, rb, l, width):
        lv = jnp.full((L,), l, jnp.int32)
        rb[pl.ds(l * D, L)] = plsc.load_gather(wbuf, [lanes, lv])
        rb[pl.ds(l * D + L, L)] = plsc.load_gather(wbuf, [lanes + L, lv])

    def write(t, rb):
        @pl.when(tbl == 0)
        def _():
            pltpu.async_copy(rb, uflat_hbm.at[pl.ds(t * CH * D, CH * D)], wsem)
        @pl.when(tbl == 1)
        def _():
            pltpu.async_copy(rb, iflat_hbm.at[pl.ds(t * CH * D, CH * D)], wsem)

    def wdrain(rb):
        pltpu.make_async_copy(uflat_hbm.at[pl.ds(0, CH * D)], rb, wsem).wait()

    read(t0, wbufA, CH)

    def step(k2, _):
        for j, (wb, wbn, rb) in enumerate(
                ((wbufA, wbufB, rbuf0), (wbufB, wbufA, rbuf1))):
            k = 2 * k2 + j
            t = t0 + k
            @pl.when(jnp.logical_and(t < NT, k < TPS))
            def _():
                @pl.when(t + 1 < t0 + TPS)
                def _():
                    @pl.when(t + 1 < NT)
                    def _():
                        read(t + 1, wbn, CH)
                rwait(wb)
                @pl.when(k >= 2)
                def _():
                    wdrain(rb)

                def body(l, _):
                    shuffle(wb, rb, l, CH)
                    return ()

                lax.fori_loop(0, CH, body, (), unroll=8)
                write(t, rb)
        return ()

    lax.fori_loop(0, (TPS + 1) // 2, step, ())
    @pl.when(t0 + 0 < NT)
    def _():
        wdrain(rbuf0)
    @pl.when(t0 + 1 < NT)
    def _():
        wdrain(rbuf1)

    @pl.when(sub == 15)
    def _():
        @pl.when(tbl == 0)
        def _():
            pltpu.sync_copy(utail_hbm, tbuf)
            pltpu.sync_copy(tbuf, uflat_hbm.at[pl.ds(NT * CH * D, TAIL * D)])
        @pl.when(tbl == 1)
        def _():
            pltpu.sync_copy(itail_hbm, tbuf)
            pltpu.sync_copy(tbuf, iflat_hbm.at[pl.ds(NT * CH * D, TAIL * D)])


@functools.partial(
    pl.kernel,
    out_type=jax.ShapeDtypeStruct((B, D), jnp.float32),
    mesh=_mesh,
    scratch_types=[
        pltpu.VMEM((2 * BPW,), jnp.int32),
        pltpu.VMEM((NCH, CH), jnp.int32),
        pltpu.VMEM((NCH, CH), jnp.int32),
        pltpu.VMEM((BPW, D), jnp.float32),
        pltpu.VMEM((BPW, D), jnp.float32),
        pltpu.SemaphoreType.DMA,
    ],
    compiler_params=pltpu.CompilerParams(use_tc_tiling_on_sc=False,
                                        needs_layout_passes=False),
)
def _gather(x_hbm, ut_hbm, it_hbm, out_hbm,
            xv, uidx, iidx, urows, irows, sem):
    wid = lax.axis_index("s") * NC + lax.axis_index("c")
    base = wid * BPW

    pltpu.sync_copy(x_hbm.at[wid], xv)

    lane = lax.iota(jnp.int32, 16)
    copies = []
    for c in range(NCH):
        for k in range(CH // L):
            flat = 2 * (c * CH + k * L) + 2 * lane
            uidx[c, pl.ds(k * L, L)] = plsc.load_gather(xv, [flat])
            iidx[c, pl.ds(k * L, L)] = plsc.load_gather(xv, [flat + 1])
        copies.append(
            pltpu.async_copy(ut_hbm.at[uidx.at[c]],
                             urows.at[pl.ds(c * CH, CH)], sem))
        copies.append(
            pltpu.async_copy(it_hbm.at[iidx.at[c]],
                             irows.at[pl.ds(c * CH, CH)], sem))
    for cp in copies:
        cp.wait()

    def body(r, _):
        for h in range(D // L):
            sl = pl.ds(h * L, L)
            urows[r, sl] = urows[r, sl] * irows[r, sl]
        return ()

    lax.fori_loop(0, BPW, body, (), unroll=4)

    pltpu.sync_copy(urows, out_hbm.at[pl.ds(base, BPW)])


def kernel(x, user_table, item_table):
    xw = x.astype(jnp.int32).reshape(NW, 2 * BPW)
    utail = user_table[NT * CH:VR].reshape(-1)
    itail = item_table[NT * CH:VR].reshape(-1)
    uflat, iflat = _relayout(user_table.T, item_table.T, utail, itail)
    ut = uflat.reshape(V, D)
    it = iflat.reshape(V, D)
    return _gather(xw, ut, it)


# speedup vs baseline: 1.0014x; 1.0014x over previous
import functools

import jax
import jax.numpy as jnp
from jax import lax
from jax.experimental import pallas as pl
from jax.experimental.pallas import tpu as pltpu
from jax.experimental.pallas import tpu_sc as plsc

NC, NS, L = 2, 16, 16
NW = NC * NS
B, D, V = 16384, 32, 1000001
BPW = B // NW
CH = 128
NCH = BPW // CH

VR = 1000000
NT = VR // CH
TAIL = VR - NT * CH
TPS = -(-NT // 16)
FLAT = V * D

_mesh = plsc.VectorSubcoreMesh(
    core_axis_name="c", subcore_axis_name="s", num_cores=NC, num_subcores=NS
)


@functools.partial(
    pl.kernel,
    out_type=(jax.ShapeDtypeStruct((FLAT,), jnp.float32),
              jax.ShapeDtypeStruct((FLAT,), jnp.float32)),
    mesh=_mesh,
    scratch_types=[
        pltpu.VMEM((D, CH + 2), jnp.float32),
        pltpu.VMEM((D, CH + 2), jnp.float32),
        pltpu.VMEM((CH * D,), jnp.float32),
        pltpu.VMEM((CH * D,), jnp.float32),
        pltpu.VMEM((TAIL * D,), jnp.float32),
        pltpu.SemaphoreType.DMA,
        pltpu.SemaphoreType.DMA,
    ],
    compiler_params=pltpu.CompilerParams(needs_layout_passes=False),
)
def _relayout(utT_hbm, itT_hbm, utail_hbm, itail_hbm, uflat_hbm, iflat_hbm,
              wbufA, wbufB, rbuf0, rbuf1, tbuf, rsem, wsem):
    wid = lax.axis_index("s") * NC + lax.axis_index("c")
    tbl = wid % 2
    sub = wid // 2
    t0 = sub * TPS

    lanes = lax.iota(jnp.int32, L)

    def read(t, wbuf, width):
        dst = wbuf.at[:, pl.ds(0, CH)]
        @pl.when(tbl == 0)
        def _():
            pltpu.async_copy(utT_hbm.at[:, pl.ds(t * CH, width)], dst, rsem)
        @pl.when(tbl == 1)
        def _():
            pltpu.async_copy(itT_hbm.at[:, pl.ds(t * CH, width)], dst, rsem)

    def rwait(wbuf):
        pltpu.make_async_copy(utT_hbm.at[:, pl.ds(0, CH)],
                              wbuf.at[:, pl.ds(0, CH)], rsem).wait()

    def shuffle(wbuf, rb, l, width):
        lv = jnp.full((L,), l, jnp.int32)
        rb[pl.ds(l * D, L)] = plsc.load_gather(wbuf, [lanes, lv])
        rb[pl.ds(l * D + L, L)] = plsc.load_gather(wbuf, [lanes + L, lv])

    def write(t, rb):
        @pl.when(tbl == 0)
        def _():
            pltpu.async_copy(rb, uflat_hbm.at[pl.ds(t * CH * D, CH * D)], wsem)
        @pl.when(tbl == 1)
        def _():
            pltpu.async_copy(rb, iflat_hbm.at[pl.ds(t * CH * D, CH * D)], wsem)

    def wdrain(rb):
        pltpu.make_async_copy(uflat_hbm.at[pl.ds(0, CH * D)], rb, wsem).wait()

    read(t0, wbufA, CH)

    def step(k2, _):
        for j, (wb, wbn, rb) in enumerate(
                ((wbufA, wbufB, rbuf0), (wbufB, wbufA, rbuf1))):
            k = 2 * k2 + j
            t = t0 + k
            @pl.when(jnp.logical_and(t < NT, k < TPS))
            def _():
                @pl.when(t + 1 < t0 + TPS)
                def _():
                    @pl.when(t + 1 < NT)
                    def _():
                        read(t + 1, wbn, CH)
                rwait(wb)
                @pl.when(k >= 2)
                def _():
                    wdrain(rb)

                def body(l, _):
                    shuffle(wb, rb, l, CH)
                    return ()

                lax.fori_loop(0, CH, body, (), unroll=8)
                write(t, rb)
        return ()

    lax.fori_loop(0, (TPS + 1) // 2, step, ())
    @pl.when(t0 + 0 < NT)
    def _():
        wdrain(rbuf0)
    @pl.when(t0 + 1 < NT)
    def _():
        wdrain(rbuf1)

    @pl.when(sub == 15)
    def _():
        @pl.when(tbl == 0)
        def _():
            pltpu.sync_copy(utail_hbm, tbuf)
            pltpu.sync_copy(tbuf, uflat_hbm.at[pl.ds(NT * CH * D, TAIL * D)])
        @pl.when(tbl == 1)
        def _():
            pltpu.sync_copy(itail_hbm, tbuf)
            pltpu.sync_copy(tbuf, iflat_hbm.at[pl.ds(NT * CH * D, TAIL * D)])


@functools.partial(
    pl.kernel,
    out_type=jax.ShapeDtypeStruct((B, D), jnp.float32),
    mesh=_mesh,
    scratch_types=[
        pltpu.VMEM((2 * BPW,), jnp.int32),
        pltpu.VMEM((NCH, CH), jnp.int32),
        pltpu.VMEM((NCH, CH), jnp.int32),
        pltpu.VMEM((BPW, D), jnp.float32),
        pltpu.VMEM((BPW, D), jnp.float32),
        pltpu.SemaphoreType.DMA,
    ],
    compiler_params=pltpu.CompilerParams(use_tc_tiling_on_sc=False,
                                        needs_layout_passes=False),
)
def _gather(x_hbm, ut_hbm, it_hbm, out_hbm,
            xv, uidx, iidx, urows, irows, sem):
    wid = lax.axis_index("s") * NC + lax.axis_index("c")
    base = wid * BPW

    pltpu.sync_copy(x_hbm.at[wid], xv)

    lane = lax.iota(jnp.int32, 16)
    copies = []
    for c in range(NCH):
        for k in range(CH // L):
            flat = 2 * (c * CH + k * L) + 2 * lane
            uidx[c, pl.ds(k * L, L)] = plsc.load_gather(xv, [flat])
            iidx[c, pl.ds(k * L, L)] = plsc.load_gather(xv, [flat + 1])
        copies.append(
            pltpu.async_copy(ut_hbm.at[uidx.at[c]],
                             urows.at[pl.ds(c * CH, CH)], sem))
        copies.append(
            pltpu.async_copy(it_hbm.at[iidx.at[c]],
                             irows.at[pl.ds(c * CH, CH)], sem))
    for cp in copies:
        cp.wait()

    def body(r, _):
        for h in range(D // L):
            sl = pl.ds(h * L, L)
            urows[r, sl] = urows[r, sl] * irows[r, sl]
        return ()

    lax.fori_loop(0, BPW, body, (), unroll=4)

    pltpu.sync_copy(urows, out_hbm.at[pl.ds(base, BPW)])


def kernel(x, user_table, item_table):
    xw = x.astype(jnp.int32).reshape(NW, 2 * BPW)
    utail = user_table[NT * CH:VR].reshape(-1)
    itail = item_table[NT * CH:VR].reshape(-1)
    uflat, iflat = _relayout(user_table.T, item_table.T, utail, itail)
    ut = uflat.reshape(V, D)
    it = iflat.reshape(V, D)
    return _gather(xw, ut, it)


# speedup vs baseline: 2.0130x; 2.0101x over previous
import functools

import jax
import jax.numpy as jnp
from jax import lax
from jax.experimental import pallas as pl
from jax.experimental.pallas import tpu as pltpu
from jax.experimental.pallas import tpu_sc as plsc

NC, NS, L = 2, 16, 16
NW = NC * NS
B, D, V = 16384, 32, 1000001
BPW = B // NW
CH = 128
NCH = BPW // CH

VR = 1000000
NT = VR // CH
TAIL = VR - NT * CH
VP = 1000008
FLAT = D * VP

NQ = 4
TPW = NT // NQ
NRUN = TPW

_mesh = plsc.VectorSubcoreMesh(
    core_axis_name="c", subcore_axis_name="s", num_cores=NC, num_subcores=NS
)


@functools.partial(
    pl.kernel,
    out_type=(jax.ShapeDtypeStruct((FLAT,), jnp.float32),
              jax.ShapeDtypeStruct((FLAT,), jnp.float32)),
    mesh=_mesh,
    scratch_types=[
        pltpu.VMEM((8, CH), jnp.float32),
        pltpu.VMEM((8, CH), jnp.float32),
        pltpu.VMEM((TAIL * D,), jnp.float32),
        pltpu.SemaphoreType.DMA,
        pltpu.SemaphoreType.DMA,
    ],
    compiler_params=pltpu.CompilerParams(needs_layout_passes=False),
)
def _detile(utT_hbm, itT_hbm, utail_hbm, itail_hbm, uflat_hbm, iflat_hbm,
            bufA, bufB, tbuf, rsem, wsem):
    wid = lax.axis_index("s") * NC + lax.axis_index("c")
    tbl = wid % 2
    w = wid // 2
    a = w % 4
    q = w // 4
    tq = q * TPW

    def read(r, buf):
        ts = tq + r
        src_u = utT_hbm.at[pl.ds(8 * a, 8), pl.ds(ts * CH, CH)]
        src_i = itT_hbm.at[pl.ds(8 * a, 8), pl.ds(ts * CH, CH)]
        @pl.when(tbl == 0)
        def _():
            pltpu.async_copy(src_u, buf, rsem)
        @pl.when(tbl == 1)
        def _():
            pltpu.async_copy(src_i, buf, rsem)

    def rwait(buf):
        pltpu.make_async_copy(utT_hbm.at[pl.ds(0, 8), pl.ds(0, CH)],
                              buf, rsem).wait()

    def write_rows(r, buf):
        ts = tq + r
        for s in range(8):
            off = (8 * a + s) * VP + ts * CH
            src = buf.at[s]
            @pl.when(tbl == 0)
            def _():
                pltpu.async_copy(src, uflat_hbm.at[pl.ds(off, CH)], wsem)
            @pl.when(tbl == 1)
            def _():
                pltpu.async_copy(src, iflat_hbm.at[pl.ds(off, CH)], wsem)

    def wdrain(buf):
        for s in range(8):
            pltpu.make_async_copy(uflat_hbm.at[pl.ds(0, CH)],
                                  buf.at[s], wsem).wait()

    read(0, bufA)

    def step(r2, _):
        for j, (bb, bn) in enumerate(((bufA, bufB), (bufB, bufA))):
            r = 2 * r2 + j
            @pl.when(r < NRUN)
            def _():
                @pl.when(r + 1 < NRUN)
                def _():
                    read(r + 1, bn)
                rwait(bb)
                @pl.when(r >= 2)
                def _():
                    wdrain(bb)
                write_rows(r, bb)
        return ()

    lax.fori_loop(0, (NRUN + 1) // 2, step, ())
    @pl.when(NRUN >= 2)
    def _():
        wdrain(bufA)
    @pl.when(NRUN >= 1)
    def _():
        wdrain(bufB)

    @pl.when(w == 15)
    def _():
        @pl.when(tbl == 0)
        def _():
            pltpu.sync_copy(utail_hbm, tbuf)
            for d in range(D):
                pltpu.sync_copy(tbuf.at[pl.ds(d * TAIL, TAIL)],
                                uflat_hbm.at[pl.ds(d * VP + NT * CH, TAIL)])
        @pl.when(tbl == 1)
        def _():
            pltpu.sync_copy(itail_hbm, tbuf)
            for d in range(D):
                pltpu.sync_copy(tbuf.at[pl.ds(d * TAIL, TAIL)],
                                iflat_hbm.at[pl.ds(d * VP + NT * CH, TAIL)])


@functools.partial(
    pl.kernel,
    out_type=jax.ShapeDtypeStruct((D, B), jnp.float32),
    mesh=_mesh,
    scratch_types=[
        pltpu.VMEM((2 * BPW,), jnp.int32),
        pltpu.VMEM((D, BPW), jnp.int32),
        pltpu.VMEM((D, BPW), jnp.int32),
        pltpu.VMEM((D, BPW), jnp.float32),
        pltpu.VMEM((D, BPW), jnp.float32),
        pltpu.SemaphoreType.DMA,
    ],
    compiler_params=pltpu.CompilerParams(needs_layout_passes=False),
)
def _lookup(x_hbm, ut_hbm, it_hbm, outT_hbm,
            xv, ubuf, ibuf, uvals, ivals, sem):
    wid = lax.axis_index("s") * NC + lax.axis_index("c")
    base = wid * BPW

    pltpu.sync_copy(x_hbm.at[wid], xv)

    lane = lax.iota(jnp.int32, 16)

    def build(h, _):
        flat = 2 * (h * L) + 2 * lane
        uv = plsc.load_gather(xv, [flat])
        iv = plsc.load_gather(xv, [flat + 1])
        sl = pl.ds(h * L, L)
        for d in range(D):
            ubuf[d, sl] = uv + d * VP
            ibuf[d, sl] = iv + d * VP
        return ()

    lax.fori_loop(0, BPW // L, build, ())

    copies = []
    for d in range(D):
        for c in range(NCH):
            sl = pl.ds(c * CH, CH)
            copies.append(
                pltpu.async_copy(ut_hbm.at[ubuf.at[d, sl]], uvals.at[d, sl], sem))
            copies.append(
                pltpu.async_copy(it_hbm.at[ibuf.at[d, sl]], ivals.at[d, sl], sem))
    for cp in copies:
        cp.wait()

    def body(h, _):
        sl = pl.ds(h * L, L)
        for d in range(D):
            uvals[d, sl] = uvals[d, sl] * ivals[d, sl]
        return ()

    lax.fori_loop(0, BPW // L, body, ())

    pltpu.sync_copy(uvals, outT_hbm.at[:, pl.ds(base, BPW)])


def kernel(x, user_table, item_table):
    xw = x.astype(jnp.int32).reshape(NW, 2 * BPW)
    utail = user_table[NT * CH:VR].T.reshape(-1)
    itail = item_table[NT * CH:VR].T.reshape(-1)
    uflat, iflat = _detile(user_table.T, item_table.T, utail, itail)
    outT = _lookup(xw, uflat, iflat)
    return outT.T
